# baseline (speedup 1.0000x reference)
import jax
import jax.numpy as jnp
from jax import lax
from jax.experimental import pallas as pl
from jax.experimental.pallas import tpu as pltpu

N = 10000
SL = 8
LN = N // SL


def _nn_kernel(pos_ref, pong_ref, out_ref):
    px = pong_ref[0]
    py = pong_ref[1]
    dx = pos_ref[0] - px
    dy = pos_ref[1] - py
    d2 = dx * dx + dy * dy
    min_val = jnp.min(d2)
    iota = (lax.broadcasted_iota(jnp.int32, (SL, LN), 0) * LN
            + lax.broadcasted_iota(jnp.int32, (SL, LN), 1))
    masked = jnp.where(d2 == min_val, iota, N)
    out_ref[0] = jnp.min(masked)


def kernel(pos_subnet_sn_xy, adj_subnet_sn_sn, ping_xy, pong_xy):
    pos_t = pos_subnet_sn_xy.T.reshape(2, SL, LN)
    out = pl.pallas_call(
        _nn_kernel,
        in_specs=[
            pl.BlockSpec(memory_space=pltpu.VMEM),
            pl.BlockSpec(memory_space=pltpu.SMEM),
        ],
        out_specs=pl.BlockSpec(memory_space=pltpu.SMEM),
        out_shape=jax.ShapeDtypeStruct((1,), jnp.int32),
    )(pos_t, pong_xy)
    return out[0]


# speedup vs baseline: 1.8282x; 1.8282x over previous
import jax
import jax.numpy as jnp
from jax import lax
from jax.experimental import pallas as pl
from jax.experimental.pallas import tpu as pltpu

N = 10000


def _nn_kernel(pos_ref, pong_ref, out_ref):
    px = pong_ref[0]
    py = pong_ref[1]
    dx = pos_ref[0:1, :] - px
    dy = pos_ref[1:2, :] - py
    d2 = dx * dx + dy * dy
    min_val = jnp.min(d2)
    iota = lax.broadcasted_iota(jnp.int32, d2.shape, 1)
    masked = jnp.where(d2 == min_val, iota, N)
    out_ref[0] = jnp.min(masked)


def kernel(pos_subnet_sn_xy, adj_subnet_sn_sn, ping_xy, pong_xy):
    pos_t = pos_subnet_sn_xy.T
    out = pl.pallas_call(
        _nn_kernel,
        in_specs=[
            pl.BlockSpec(memory_space=pltpu.VMEM),
            pl.BlockSpec(memory_space=pltpu.SMEM),
        ],
        out_specs=pl.BlockSpec(memory_space=pltpu.SMEM),
        out_shape=jax.ShapeDtypeStruct((1,), jnp.int32),
    )(pos_t, pong_xy)
    return out[0]
